# baseline (speedup 1.0000x reference)
import functools

import jax
import jax.numpy as jnp
from jax import lax
from jax.experimental import pallas as pl
from jax.experimental.pallas import tpu as pltpu
from jax.experimental.pallas import tpu_sc as plsc

_SEQ_BLOCK = 2048


def _tc_body(x_ref, emb_ref, out_ref):
    out_ref[0] = x_ref[0] + emb_ref[...]


def _tc_add(x, embedding, seq_block, n_seq_blocks):
    batch, seq_len, dim = x.shape
    grid = (n_seq_blocks, batch)
    return pl.pallas_call(
        _tc_body,
        grid=grid,
        in_specs=[
            pl.BlockSpec((1, seq_block, dim), lambda s, b: (b, s, 0)),
            pl.BlockSpec((seq_block, dim), lambda s, b: (s, 0)),
        ],
        out_specs=pl.BlockSpec((1, seq_block, dim), lambda s, b: (b, s, 0)),
        out_shape=jax.ShapeDtypeStruct((batch, seq_len, dim), x.dtype),
        compiler_params=pltpu.CompilerParams(
            vmem_limit_bytes=64 * 1024 * 1024,
        ),
    )(x, embedding)


def _tc_body3(x_ref, emb_ref, partial_hbm_ref, out_ref):
    del partial_hbm_ref
    out_ref[0] = x_ref[0] + emb_ref[...]


def _tc_add_into(x, embedding, partial_out, seq_block, n_seq_blocks):
    batch, seq_len, dim = x.shape
    grid = (n_seq_blocks, batch)
    return pl.pallas_call(
        _tc_body3,
        grid=grid,
        in_specs=[
            pl.BlockSpec((1, seq_block, dim), lambda s, b: (b, s, 0)),
            pl.BlockSpec((seq_block, dim), lambda s, b: (s, 0)),
            pl.BlockSpec(memory_space=pl.ANY),
        ],
        out_specs=pl.BlockSpec((1, seq_block, dim), lambda s, b: (b, s, 0)),
        out_shape=jax.ShapeDtypeStruct((batch, seq_len, dim), x.dtype),
        input_output_aliases={2: 0},
        compiler_params=pltpu.CompilerParams(
            vmem_limit_bytes=64 * 1024 * 1024,
        ),
    )(x, embedding, partial_out)


_CHUNK_ROWS = 16
_N_XBUF = 3
_N_EBUF = 2


def _sc_add(x, embedding, seq_lo=0, full_out=True):
    n_batch, seq_len, dim = x.shape
    span = seq_len - seq_lo
    info = plsc.get_sparse_core_info()
    nw = info.num_cores * info.num_subcores
    rows_w = span // nw
    n_chunks = rows_w // _CHUNK_ROWS
    n_steps = n_chunks * n_batch
    mesh = plsc.VectorSubcoreMesh(core_axis_name="c", subcore_axis_name="s")

    @functools.partial(
        pl.kernel,
        mesh=mesh,
        out_type=jax.ShapeDtypeStruct(
            (n_batch, seq_len if full_out else span, dim), x.dtype
        ),
        scratch_types=[
            [pltpu.VMEM((_CHUNK_ROWS, dim), jnp.float32) for _ in range(_N_XBUF)],
            [pltpu.VMEM((_CHUNK_ROWS, dim), jnp.float32) for _ in range(_N_EBUF)],
            [pltpu.SemaphoreType.DMA for _ in range(_N_XBUF)],
            [pltpu.SemaphoreType.DMA for _ in range(_N_EBUF)],
            [pltpu.SemaphoreType.DMA for _ in range(_N_XBUF)],
        ],
    )
    def k(x_hbm, emb_hbm, out_hbm, xbufs, ebufs, xsems, esems, osems):
        wid = lax.axis_index("s") * info.num_cores + lax.axis_index("c")
        base_row = wid * rows_w

        def row0(ci):
            return pl.multiple_of(base_row + ci * _CHUNK_ROWS, 8)

        def start_xload(step):
            ci, b = divmod(step, n_batch)
            j = step % _N_XBUF
            return pltpu.async_copy(
                x_hbm.at[b, pl.ds(pl.multiple_of(seq_lo + row0(ci), 8), _CHUNK_ROWS), :],
                xbufs[j],
                xsems[j],
            )

        def start_eload(ci):
            j = ci % _N_EBUF
            return pltpu.async_copy(
                emb_hbm.at[pl.ds(pl.multiple_of(seq_lo + row0(ci), 8), _CHUNK_ROWS), :],
                ebufs[j],
                esems[j],
            )

        def start_store(step):
            ci, b = divmod(step, n_batch)
            j = step % _N_XBUF
            out_row = (seq_lo + row0(ci)) if full_out else row0(ci)
            return pltpu.async_copy(
                xbufs[j],
                out_hbm.at[b, pl.ds(pl.multiple_of(out_row, 8), _CHUNK_ROWS), :],
                osems[j],
            )

        loads = {0: start_xload(0)}
        eloads = {0: start_eload(0)}
        stores = {}
        for i in range(n_steps):
            ci, b = divmod(i, n_batch)
            nxt = i + 1
            if nxt < n_steps:
                prev = nxt - _N_XBUF
                if prev in stores:
                    stores.pop(prev).wait()
                loads[nxt] = start_xload(nxt)
                nci = nxt // n_batch
                if nci != ci and nci not in eloads:
                    eloads[nci] = start_eload(nci)
            if b == 0:
                eloads.pop(ci).wait()
            loads.pop(i).wait()
            xb = xbufs[i % _N_XBUF]
            eb = ebufs[ci % _N_EBUF]

            def row_body(r, carry):
                @plsc.parallel_loop(0, dim, step=16, unroll=8)
                def _(j):
                    sl = pl.ds(pl.multiple_of(j, 16), 16)
                    xb[r, sl] = xb[r, sl] + eb[r, sl]

                return carry

            lax.fori_loop(0, _CHUNK_ROWS, row_body, 0)

            stores[i] = start_store(i)
        for s in stores.values():
            s.wait()

    return k(x, embedding)


_SC_SEQ_ROWS = 512


def kernel(x, embedding):
    batch, seq_len, dim = x.shape
    seq_lo = seq_len - _SC_SEQ_ROWS
    sc_out = _sc_add(x, embedding, seq_lo=seq_lo, full_out=False)
    tc_out = _tc_add(x, embedding, seq_block=1920, n_seq_blocks=seq_lo // 1920)
    return lax.dynamic_update_slice(tc_out, sc_out, (0, seq_lo, 0))


# speedup vs baseline: 1.0053x; 1.0053x over previous
import functools

import jax
import jax.numpy as jnp
from jax import lax
from jax.experimental import pallas as pl
from jax.experimental.pallas import tpu as pltpu
from jax.experimental.pallas import tpu_sc as plsc

def _tc_body3(x_ref, emb_ref, partial_hbm_ref, out_ref):
    del partial_hbm_ref
    out_ref[0] = x_ref[0] + emb_ref[...]


def _tc_add_into(x, embedding, partial_out, seq_block, n_seq_blocks):
    batch, seq_len, dim = x.shape
    grid = (n_seq_blocks, batch)
    return pl.pallas_call(
        _tc_body3,
        grid=grid,
        in_specs=[
            pl.BlockSpec((1, seq_block, dim), lambda s, b: (b, s, 0)),
            pl.BlockSpec((seq_block, dim), lambda s, b: (s, 0)),
            pl.BlockSpec(memory_space=pl.ANY),
        ],
        out_specs=pl.BlockSpec((1, seq_block, dim), lambda s, b: (b, s, 0)),
        out_shape=jax.ShapeDtypeStruct((batch, seq_len, dim), x.dtype),
        input_output_aliases={2: 0},
        compiler_params=pltpu.CompilerParams(
            vmem_limit_bytes=64 * 1024 * 1024,
        ),
    )(x, embedding, partial_out)


_CHUNK_ROWS = 16
_N_XBUF = 3
_N_EBUF = 2


def _sc_add(x, embedding, seq_lo=0):
    n_batch, seq_len, dim = x.shape
    span = seq_len - seq_lo
    info = plsc.get_sparse_core_info()
    nw = info.num_cores * info.num_subcores
    rows_w = span // nw
    n_chunks = rows_w // _CHUNK_ROWS
    n_steps = n_chunks * n_batch
    mesh = plsc.VectorSubcoreMesh(core_axis_name="c", subcore_axis_name="s")

    @functools.partial(
        pl.kernel,
        mesh=mesh,
        out_type=jax.ShapeDtypeStruct((n_batch, seq_len, dim), x.dtype),
        scratch_types=[
            [pltpu.VMEM((_CHUNK_ROWS, dim), jnp.float32) for _ in range(_N_XBUF)],
            [pltpu.VMEM((_CHUNK_ROWS, dim), jnp.float32) for _ in range(_N_EBUF)],
            [pltpu.SemaphoreType.DMA for _ in range(_N_XBUF)],
            [pltpu.SemaphoreType.DMA for _ in range(_N_EBUF)],
            [pltpu.SemaphoreType.DMA for _ in range(_N_XBUF)],
        ],
    )
    def k(x_hbm, emb_hbm, out_hbm, xbufs, ebufs, xsems, esems, osems):
        wid = lax.axis_index("s") * info.num_cores + lax.axis_index("c")
        base_row = wid * rows_w

        def row0(ci):
            return pl.multiple_of(base_row + ci * _CHUNK_ROWS, 8)

        def start_xload(step):
            ci, b = divmod(step, n_batch)
            j = step % _N_XBUF
            return pltpu.async_copy(
                x_hbm.at[b, pl.ds(pl.multiple_of(seq_lo + row0(ci), 8), _CHUNK_ROWS), :],
                xbufs[j],
                xsems[j],
            )

        def start_eload(ci):
            j = ci % _N_EBUF
            return pltpu.async_copy(
                emb_hbm.at[pl.ds(pl.multiple_of(seq_lo + row0(ci), 8), _CHUNK_ROWS), :],
                ebufs[j],
                esems[j],
            )

        def start_store(step):
            ci, b = divmod(step, n_batch)
            j = step % _N_XBUF
            out_row = seq_lo + row0(ci)
            return pltpu.async_copy(
                xbufs[j],
                out_hbm.at[b, pl.ds(pl.multiple_of(out_row, 8), _CHUNK_ROWS), :],
                osems[j],
            )

        loads = {0: start_xload(0)}
        eloads = {0: start_eload(0)}
        stores = {}
        for i in range(n_steps):
            ci, b = divmod(i, n_batch)
            nxt = i + 1
            if nxt < n_steps:
                prev = nxt - _N_XBUF
                if prev in stores:
                    stores.pop(prev).wait()
                loads[nxt] = start_xload(nxt)
                nci = nxt // n_batch
                if nci != ci and nci not in eloads:
                    eloads[nci] = start_eload(nci)
            if b == 0:
                eloads.pop(ci).wait()
            loads.pop(i).wait()
            xb = xbufs[i % _N_XBUF]
            eb = ebufs[ci % _N_EBUF]

            def row_body(r, carry):
                @plsc.parallel_loop(0, dim, step=16, unroll=8)
                def _(j):
                    sl = pl.ds(pl.multiple_of(j, 16), 16)
                    xb[r, sl] = xb[r, sl] + eb[r, sl]

                return carry

            lax.fori_loop(0, _CHUNK_ROWS, row_body, 0)

            stores[i] = start_store(i)
        for s in stores.values():
            s.wait()

    return k(x, embedding)


_SC_SEQ_ROWS = 512


def kernel(x, embedding):
    batch, seq_len, dim = x.shape
    seq_lo = seq_len - _SC_SEQ_ROWS
    sc_full = _sc_add(x, embedding, seq_lo=seq_lo)
    return _tc_add_into(
        x, embedding, sc_full, seq_block=1920, n_seq_blocks=seq_lo // 1920
    )
